# speedup vs baseline: 7.3860x; 7.3860x over previous
import jax
import jax.numpy as jnp
from jax.experimental import pallas as pl

_C = 32000
_PAD = 0
_SM = 0.1
_CONF = 1.0 - _SM
_EPS = _SM / (_C - 2)
_BR = 64


def _body(t_ref, x_ref, o_ref):
    x = x_ref[...]
    t = t_ref[0, 0, :]
    m = jnp.max(x, axis=1, keepdims=True)
    s = jnp.sum(jnp.exp(x - m), axis=1)
    z = m[:, 0] + jnp.log(s)
    sp = jnp.sum(x, axis=1)
    cols = jax.lax.broadcasted_iota(jnp.int32, x.shape, 1)
    pt = jnp.sum(jnp.where(cols == t[:, None], x, 0.0), axis=1)
    p0 = x[:, 0]
    lt = pt - z
    l0 = p0 - z
    srow = sp - _C * z
    loss = -(_CONF * lt + _EPS * (srow - l0 - lt))
    o_ref[0, 0, :] = jnp.where(t == _PAD, 0.0, loss)


def kernel(pred, target):
    n = pred.shape[0]
    nb = n // _BR
    t3 = target.astype(jnp.int32).reshape(nb, 1, _BR)
    rows = pl.pallas_call(
        _body,
        grid=(nb,),
        in_specs=[
            pl.BlockSpec((1, 1, _BR), lambda i: (i, 0, 0)),
            pl.BlockSpec((_BR, _C), lambda i: (i, 0)),
        ],
        out_specs=pl.BlockSpec((1, 1, _BR), lambda i: (i, 0, 0)),
        out_shape=jax.ShapeDtypeStruct((nb, 1, _BR), jnp.float32),
    )(t3, pred)
    return jnp.mean(rows.reshape(n))
